# Initial kernel scaffold; baseline (speedup 1.0000x reference)
import jax
import jax.numpy as jnp
from jax.experimental import pallas as pl


def kernel(inputs, W_emb, b_emb, W0, b0, W1, b1, W2, b2, W3, b3):
    raise NotImplementedError("write your pallas kernel here")


# speedup vs baseline: 9.2954x; 9.2954x over previous
import jax
import jax.numpy as jnp
from jax import lax
from jax.experimental import pallas as pl
from jax.experimental.pallas import tpu as pltpu
from jax.experimental.pallas import tpu_sc as plsc

N = 4096
C = 128
K = 16
RT = 512
GRID = N // RT
NW = 32
NPT = N // NW
GCH = 8
NCH = NPT // GCH
F32 = jnp.float32


def _embed_body(x_ref, w_ref, b_ref, o_ref):
    o_ref[...] = jnp.maximum(
        jnp.dot(x_ref[...], w_ref[...], preferred_element_type=F32)
        + b_ref[...], 0.0)


def _embed(x, w, b):
    return pl.pallas_call(
        _embed_body,
        grid=(GRID,),
        in_specs=[
            pl.BlockSpec((RT, x.shape[1]), lambda i: (i, 0)),
            pl.BlockSpec((x.shape[1], C), lambda i: (0, 0)),
            pl.BlockSpec((1, C), lambda i: (0, 0)),
        ],
        out_specs=pl.BlockSpec((RT, C), lambda i: (i, 0)),
        out_shape=jax.ShapeDtypeStruct((N, C), F32),
    )(x, w, b.reshape(1, C))


def _knn_body(x_ref, xt_ref, xsqc_ref, xsqr_ref, idx_ref):
    g = jnp.dot(x_ref[...], xt_ref[...], preferred_element_type=F32)
    d = (xsqc_ref[...] - 2.0 * g) + xsqr_ref[...]
    iota = lax.broadcasted_iota(jnp.int32, (RT, N), 1)
    cols = []
    for _ in range(K):
        m = jnp.min(d, axis=1, keepdims=True)
        am = jnp.min(jnp.where(d == m, iota, N), axis=1, keepdims=True)
        cols.append(am)
        d = jnp.where(iota == am, jnp.inf, d)
    idx_ref[...] = jnp.concatenate(cols, axis=1)


def _knn(x, xsq):
    return pl.pallas_call(
        _knn_body,
        grid=(GRID,),
        in_specs=[
            pl.BlockSpec((RT, C), lambda i: (i, 0)),
            pl.BlockSpec((C, N), lambda i: (0, 0)),
            pl.BlockSpec((RT, 1), lambda i: (i, 0)),
            pl.BlockSpec((1, N), lambda i: (0, 0)),
        ],
        out_specs=pl.BlockSpec((RT, K), lambda i: (i, 0)),
        out_shape=jax.ShapeDtypeStruct((N, K), jnp.int32),
    )(x, x.T, xsq.reshape(N, 1), xsq.reshape(1, N))


def _edge_body(x_ref, xj_ref, wt_ref, wb_ref, b_ref, r_ref, o_ref):
    xi = x_ref[...]
    a = jnp.dot(xi, wt_ref[...], preferred_element_type=F32) + b_ref[...]
    acc = None
    for j in range(K):
        e = xj_ref[:, j, :] - xi
        h = jnp.maximum(
            a + jnp.dot(e, wb_ref[...], preferred_element_type=F32), 0.0)
        acc = h if acc is None else jnp.maximum(acc, h)
    o_ref[...] = acc + r_ref[...]


def _edge(x, xj, wt, wb, b, res):
    return pl.pallas_call(
        _edge_body,
        grid=(GRID,),
        in_specs=[
            pl.BlockSpec((RT, C), lambda i: (i, 0)),
            pl.BlockSpec((RT, K, C), lambda i: (i, 0, 0)),
            pl.BlockSpec((C, C), lambda i: (0, 0)),
            pl.BlockSpec((C, C), lambda i: (0, 0)),
            pl.BlockSpec((1, C), lambda i: (0, 0)),
            pl.BlockSpec((RT, C), lambda i: (i, 0)),
        ],
        out_specs=pl.BlockSpec((RT, C), lambda i: (i, 0)),
        out_shape=jax.ShapeDtypeStruct((N, C), F32),
    )(x, xj, wt, wb, b.reshape(1, C), res)


def _sc_gather_body(idx_hbm, x_hbm, out_hbm, idxv, rows, sem):
    wid = lax.axis_index("s") * 2 + lax.axis_index("c")
    pltpu.sync_copy(idx_hbm.at[wid], idxv)

    def chunk(g, _):
        pltpu.async_copy(x_hbm.at[idxv.at[g]], rows, sem).wait()
        base = (wid * NPT + g * GCH) * K
        pltpu.sync_copy(rows, out_hbm.at[pl.ds(base, GCH * K)])
        return 0

    lax.fori_loop(0, NCH, chunk, 0)


def _sc_gather(idx, x):
    mesh = plsc.VectorSubcoreMesh(core_axis_name="c", subcore_axis_name="s")
    k = pl.kernel(
        _sc_gather_body,
        out_type=jax.ShapeDtypeStruct((N * K, C), F32),
        mesh=mesh,
        scratch_types=[
            pltpu.VMEM((NCH, GCH * K), jnp.int32),
            pltpu.VMEM((GCH * K, C), F32),
            pltpu.SemaphoreType.DMA,
        ],
    )
    return k(idx.reshape(NW, NCH, GCH * K), x)


def kernel(inputs, W_emb, b_emb, W0, b0, W1, b1, W2, b2, W3, b3):
    x = _embed(inputs, W_emb, b_emb)
    feas = []
    zero = jnp.zeros((N, C), F32)
    for i, (W, b) in enumerate([(W0, b0), (W1, b1), (W2, b2), (W3, b3)]):
        xsq = jnp.sum(x * x, axis=-1)
        idx = _knn(x, xsq)
        xj = _sc_gather(idx, x).reshape(N, K, C)
        res = zero if i == 0 else x
        x = _edge(x, xj, W[:C, :], W[C:, :], b, res)
        feas.append(x)
    return jnp.concatenate(feas, axis=-1)


# speedup vs baseline: 10.6014x; 1.1405x over previous
import functools

import jax
import jax.numpy as jnp
from jax import lax
from jax.experimental import pallas as pl
from jax.experimental.pallas import tpu as pltpu
from jax.experimental.pallas import tpu_sc as plsc
from jax.sharding import Mesh, PartitionSpec as P
from jax.experimental.shard_map import shard_map

N = 4096
C = 128
K = 16
RT = 512
NW = 32
GCH = 8
F32 = jnp.float32


def _embed_body(x_ref, w_ref, b_ref, o_ref):
    o_ref[...] = jnp.maximum(
        jnp.dot(x_ref[...], w_ref[...], preferred_element_type=F32)
        + b_ref[...], 0.0)


def _embed(x, w, b):
    nl, cin = x.shape
    return pl.pallas_call(
        _embed_body,
        grid=(nl // RT,),
        in_specs=[
            pl.BlockSpec((RT, cin), lambda i: (i, 0)),
            pl.BlockSpec((cin, C), lambda i: (0, 0)),
            pl.BlockSpec((1, C), lambda i: (0, 0)),
        ],
        out_specs=pl.BlockSpec((RT, C), lambda i: (i, 0)),
        out_shape=jax.ShapeDtypeStruct((nl, C), F32),
    )(x, w, b.reshape(1, C))


def _knn_body(x_ref, xt_ref, xsqc_ref, xsqr_ref, idx_ref):
    g = jnp.dot(x_ref[...], xt_ref[...], preferred_element_type=F32)
    d = (xsqc_ref[...] - 2.0 * g) + xsqr_ref[...]
    iota = lax.broadcasted_iota(jnp.int32, (RT, N), 1)
    cols = []
    for _ in range(K):
        m = jnp.min(d, axis=1, keepdims=True)
        am = jnp.min(jnp.where(d == m, iota, N), axis=1, keepdims=True)
        cols.append(am)
        d = jnp.where(iota == am, jnp.inf, d)
    idx_ref[...] = jnp.concatenate(cols, axis=1)


def _knn(xl, xf, xsql, xsqf):
    nl = xl.shape[0]
    return pl.pallas_call(
        _knn_body,
        grid=(nl // RT,),
        in_specs=[
            pl.BlockSpec((RT, C), lambda i: (i, 0)),
            pl.BlockSpec((C, N), lambda i: (0, 0)),
            pl.BlockSpec((RT, 1), lambda i: (i, 0)),
            pl.BlockSpec((1, N), lambda i: (0, 0)),
        ],
        out_specs=pl.BlockSpec((RT, K), lambda i: (i, 0)),
        out_shape=jax.ShapeDtypeStruct((nl, K), jnp.int32),
    )(xl, xf.T, xsql.reshape(nl, 1), xsqf.reshape(1, N))


def _edge_body(x_ref, xj_ref, wt_ref, wb_ref, b_ref, r_ref, o_ref):
    xi = x_ref[...]
    a = jnp.dot(xi, wt_ref[...], preferred_element_type=F32) + b_ref[...]
    acc = None
    for j in range(K):
        e = xj_ref[:, j, :] - xi
        h = jnp.maximum(
            a + jnp.dot(e, wb_ref[...], preferred_element_type=F32), 0.0)
        acc = h if acc is None else jnp.maximum(acc, h)
    o_ref[...] = acc + r_ref[...]


def _edge(xl, xj, wt, wb, b, res):
    nl = xl.shape[0]
    return pl.pallas_call(
        _edge_body,
        grid=(nl // RT,),
        in_specs=[
            pl.BlockSpec((RT, C), lambda i: (i, 0)),
            pl.BlockSpec((RT, K, C), lambda i: (i, 0, 0)),
            pl.BlockSpec((C, C), lambda i: (0, 0)),
            pl.BlockSpec((C, C), lambda i: (0, 0)),
            pl.BlockSpec((1, C), lambda i: (0, 0)),
            pl.BlockSpec((RT, C), lambda i: (i, 0)),
        ],
        out_specs=pl.BlockSpec((RT, C), lambda i: (i, 0)),
        out_shape=jax.ShapeDtypeStruct((nl, C), F32),
    )(xl, xj, wt, wb, b.reshape(1, C), res)


def _sc_gather_body(npt, nch, idx_hbm, x_hbm, out_hbm, idxv, rows, sem):
    wid = lax.axis_index("s") * 2 + lax.axis_index("c")
    pltpu.sync_copy(idx_hbm.at[wid], idxv)

    def chunk(g, _):
        pltpu.async_copy(x_hbm.at[idxv.at[g]], rows, sem).wait()
        base = (wid * npt + g * GCH) * K
        pltpu.sync_copy(rows, out_hbm.at[pl.ds(base, GCH * K)])
        return 0

    lax.fori_loop(0, nch, chunk, 0)


def _sc_gather(idx, xf):
    nl = idx.shape[0]
    npt = nl // NW
    nch = npt // GCH
    mesh = plsc.VectorSubcoreMesh(core_axis_name="c", subcore_axis_name="s")
    k = pl.kernel(
        functools.partial(_sc_gather_body, npt, nch),
        out_type=jax.ShapeDtypeStruct((nl * K, C), F32),
        mesh=mesh,
        scratch_types=[
            pltpu.VMEM((nch, GCH * K), jnp.int32),
            pltpu.VMEM((GCH * K, C), F32),
            pltpu.SemaphoreType.DMA,
        ],
    )
    return k(idx.reshape(NW, nch, GCH * K), xf)


def _run_shard(nd, inputs, W_emb, b_emb, Ws, bs):
    nl = N // nd
    did = lax.axis_index("d")
    x = _embed(inputs, W_emb, b_emb)
    feas = []
    zero = jnp.zeros((nl, C), F32)
    for i, (W, b) in enumerate(zip(Ws, bs)):
        xf = lax.all_gather(x, "d", tiled=True)
        xsqf = jnp.sum(xf * xf, axis=-1)
        xsql = lax.dynamic_slice(xsqf, (did * nl,), (nl,))
        idx = _knn(x, xf, xsql, xsqf)
        xj = _sc_gather(idx, xf).reshape(nl, K, C)
        res = zero if i == 0 else x
        x = _edge(x, xj, W[:C, :], W[C:, :], b, res)
        feas.append(x)
    return jnp.concatenate(feas, axis=-1)


def kernel(inputs, W_emb, b_emb, W0, b0, W1, b1, W2, b2, W3, b3):
    devs = jax.devices()
    nd = 2 if len(devs) >= 2 else 1
    mesh = Mesh(devs[:nd], ("d",))
    run = shard_map(
        functools.partial(_run_shard, nd),
        mesh=mesh,
        in_specs=(P("d"), P(None, None), P(None),
                  (P(None, None),) * 4, (P(None),) * 4),
        out_specs=P("d", None),
        check_rep=False,
    )
    return run(inputs, W_emb, b_emb, (W0, W1, W2, W3), (b0, b1, b2, b3))
